# baseline (speedup 1.0000x reference)
import functools

import jax
import jax.numpy as jnp
from jax import lax
from jax.experimental import pallas as pl
from jax.experimental.pallas import tpu as pltpu
from jax.experimental.pallas import tpu_sc as plsc

D_ITEM = 64
D_OTHER = 16
_CH = 256
_HC = 128


def _item_transform(item_table, wfi):
    V = item_table.shape[0]
    BR = 2048
    grid = (V + BR - 1) // BR

    def body(t_ref, w_ref, o_ref):
        o_ref[...] = lax.dot_general(t_ref[...], w_ref[...],
                                     (((1,), (1,)), ((), ())),
                                     preferred_element_type=jnp.float32)

    return pl.pallas_call(
        body,
        grid=(grid,),
        in_specs=[pl.BlockSpec((BR, D_ITEM), lambda i: (i, 0)),
                  pl.BlockSpec((D_ITEM, D_ITEM), lambda i: (0, 0))],
        out_specs=pl.BlockSpec((BR, D_ITEM), lambda i: (i, 0)),
        out_shape=jax.ShapeDtypeStruct((V, D_ITEM), jnp.float32),
    )(item_table, wfi)


def _pack_cols(m):
    lo = jnp.concatenate([m[:, 0:16], m[:, 32:48]], axis=1)
    hi = jnp.concatenate([m[:, 16:32], m[:, 48:64]], axis=1)
    lo16 = lax.bitcast_convert_type(lo.astype(jnp.bfloat16), jnp.uint16)
    hi16 = lax.bitcast_convert_type(hi.astype(jnp.bfloat16), jnp.uint16)
    word = lo16.astype(jnp.uint32) | (hi16.astype(jnp.uint32) << 16)
    return lax.bitcast_convert_type(word, jnp.int32)


def _small_transforms(brand_table, cat_table, wfb, wfc, wfp, Wp, bp2, bf2):
    def body(bt, ct, wb, wc, wpf, wpp, bpr, bfr, ob, oc, opv):
        b2 = lax.dot_general(bt[...], wb[...], (((1,), (1,)), ((), ())),
                             preferred_element_type=jnp.float32)
        ob[...] = _pack_cols(b2)
        c = lax.dot_general(bpr[...], wpf[...], (((1,), (1,)), ((), ())),
                            preferred_element_type=jnp.float32) + bfr[...]
        c2 = lax.dot_general(ct[...], wc[...], (((1,), (1,)), ((), ())),
                             preferred_element_type=jnp.float32) + c
        oc[...] = _pack_cols(c2)
        opv[...] = lax.dot_general(wpp[...], wpf[...], (((0,), (1,)), ((), ())),
                                   preferred_element_type=jnp.float32)

    nb = brand_table.shape[0]
    nc = cat_table.shape[0]
    return pl.pallas_call(
        body,
        out_shape=[jax.ShapeDtypeStruct((nb, 32), jnp.int32),
                   jax.ShapeDtypeStruct((nc, 32), jnp.int32),
                   jax.ShapeDtypeStruct((1, D_ITEM), jnp.float32)],
    )(brand_table, cat_table, wfb, wfc, wfp, Wp, bp2, bf2)


def _sc_fuse(packed, item2, brand2p, cat2p, pv, n, nb, nc):
    info = plsc.get_sparse_core_info()
    nw = info.num_cores * info.num_subcores
    assert n % (nw * _CH) == 0
    rows_w = n // nw
    nch = rows_w // _CH
    assert nch % 2 == 0
    npairs = nch // 2
    mesh = plsc.VectorSubcoreMesh(core_axis_name="c", subcore_axis_name="s")

    @functools.partial(
        pl.kernel, mesh=mesh,
        compiler_params=pltpu.CompilerParams(use_tc_tiling_on_sc=False,
                                             needs_layout_passes=False),
        out_type=jax.ShapeDtypeStruct((n // 2, 2 * D_ITEM), jnp.float32),
        scratch_types=[
            pltpu.VMEM((8, _HC), jnp.int32),
            pltpu.VMEM((8, _HC), jnp.int32),
            pltpu.VMEM((_CH, D_ITEM), jnp.float32),
            pltpu.VMEM((_CH, D_ITEM), jnp.float32),
            pltpu.VMEM((_CH // 2, 2 * D_ITEM), jnp.float32),
            pltpu.VMEM((_CH // 2, 2 * D_ITEM), jnp.float32),
            pltpu.VMEM((nb, 32), jnp.int32),
            pltpu.VMEM((nc, 32), jnp.int32),
            pltpu.VMEM((D_ITEM,), jnp.float32),
            pltpu.SemaphoreType.DMA,
            pltpu.SemaphoreType.DMA,
            pltpu.SemaphoreType.DMA,
            pltpu.SemaphoreType.DMA,
            pltpu.SemaphoreType.DMA,
            pltpu.SemaphoreType.DMA,
        ],
    )
    def k(packed_hbm, it2, br2_hbm, ct2_hbm, pv_hbm, out_hbm,
          xb0, xb1, a0, a1, s0buf, s1buf, br_v, ct_v, pv_v,
          si0, si1, sg0, sg1, so0, so1):
        c16 = [jnp.full((16,), r, jnp.int32) for r in range(16)]
        col16 = [jnp.arange(16, dtype=jnp.int32) + 16 * w for w in range(2)]
        wid = lax.axis_index("s") * info.num_cores + lax.axis_index("c")
        cbase = wid * nch
        rbase = wid * rows_w
        pltpu.sync_copy(pv_hbm.at[0], pv_v)
        pltpu.sync_copy(br2_hbm, br_v)
        pltpu.sync_copy(ct2_hbm, ct_v)
        xb = (xb0, xb1)
        A = (a0, a1)
        S = (s0buf, s1buf)
        si = (si0, si1)
        sg = (sg0, sg1)
        so = (so0, so1)

        def fire_idx(g, s):
            pltpu.async_copy(packed_hbm.at[cbase + g], xb[s], si[s])

        def wait_idx(s):
            pltpu.make_async_copy(packed_hbm.at[0], xb[s], si[s]).wait()

        def fire_gathers(g, s):
            for h in range(2):
                dst = pl.ds(h * _HC, _HC)
                pltpu.async_copy(it2.at[xb[s].at[h]], A[s].at[dst], sg[s])

        def wait_gathers(s):
            for h in range(2):
                dst = pl.ds(h * _HC, _HC)
                pltpu.make_async_copy(it2.at[pl.ds(0, _HC)], A[s].at[dst],
                                      sg[s]).wait()

        def fire_store(g, s):
            pltpu.async_copy(
                S[s], out_hbm.at[pl.ds((rbase + g * _CH) // 2, _CH // 2)],
                so[s])

        def wait_store(s):
            pltpu.make_async_copy(S[s], out_hbm.at[pl.ds(0, _CH // 2)],
                                  so[s]).wait()

        pvs_slices = [pl.ds(t * 16, 16) for t in range(4)]

        def gather_packed(tab, rowb):
            out = []
            for w in range(2):
                words = plsc.load_gather(tab, [rowb, col16[w]])
                s0, s1 = plsc.unpack(plsc.bitcast(words, jnp.bfloat16),
                                     format=plsc.PackFormat.INTERLEAVED)
                out.append(s0)
                out.append(s1)
            return out

        def alu(s):
            av, sv, xv = A[s], S[s], xb[s]
            pvs = [pv_v[sl] for sl in pvs_slices]

            def grp(j, carry2):
                jhi = j // 8
                jlo = pl.ds((j % 8) * 16, 16)
                bi_vec = xv[2 + jhi, jlo]
                ci_vec = xv[4 + jhi, jlo]
                pr_vec = plsc.bitcast(xv[6 + jhi, jlo], jnp.float32)
                for r in range(16):
                    nr = j * 16 + r
                    rb = bi_vec.at[c16[r]].get(mode="promise_in_bounds")
                    rc = ci_vec.at[c16[r]].get(mode="promise_in_bounds")
                    pb = pr_vec.at[c16[r]].get(mode="promise_in_bounds")
                    bs = gather_packed(br_v, rb)
                    cs = gather_packed(ct_v, rc)
                    for t in range(4):
                        sl = pvs_slices[t]
                        osl = pl.ds((r % 2) * D_ITEM + 16 * t, 16)
                        sv[8 * j + r // 2, osl] = (av[nr, sl] + bs[t] + cs[t]
                                                   + pb * pvs[t])
                return carry2

            lax.fori_loop(0, _CH // 16, grp, 0)

        fire_idx(0, 0)
        fire_idx(1, 1)
        wait_idx(0)
        fire_gathers(0, 0)

        def pair(p, carry):
            g = 2 * p
            wait_gathers(0)

            @pl.when(p > 0)
            def _():
                wait_store(1)

            wait_idx(1)
            fire_gathers(g + 1, 1)
            alu(0)
            fire_store(g, 0)

            @pl.when(p < npairs - 1)
            def _():
                fire_idx(g + 2, 0)

            wait_gathers(1)
            wait_store(0)

            @pl.when(p < npairs - 1)
            def _():
                wait_idx(0)
                fire_gathers(g + 2, 0)

            alu(1)
            fire_store(g + 1, 1)

            @pl.when(p < npairs - 1)
            def _():
                fire_idx(g + 3, 1)

            return carry

        lax.fori_loop(0, npairs, pair, 0)
        wait_store(1)

    return k(packed, item2, brand2p, cat2p, pv)


def kernel(x, item_table, brand_table, cat_table, Wp, bp, Wf, bf):
    n = x.shape[0]
    nchunks = n // _CH
    ii = x[:, 0].astype(jnp.int32).reshape(nchunks, 2, _HC)
    bi = x[:, 1].astype(jnp.int32).reshape(nchunks, 2, _HC)
    ci = x[:, 2].astype(jnp.int32).reshape(nchunks, 2, _HC)
    pb = lax.bitcast_convert_type(x[:, 3], jnp.int32).reshape(nchunks, 2, _HC)
    packed = jnp.concatenate([ii, bi, ci, pb], axis=1)
    wfi = Wf[:, :D_ITEM]
    wfb = Wf[:, D_ITEM:D_ITEM + D_OTHER]
    wfc = Wf[:, D_ITEM + D_OTHER:D_ITEM + 2 * D_OTHER]
    wfp = Wf[:, D_ITEM + 2 * D_OTHER:]
    item2 = _item_transform(item_table, wfi)
    brand2p, cat2p, pv = _small_transforms(
        brand_table, cat_table, wfb, wfc, wfp, Wp,
        bp.reshape(1, -1), bf.reshape(1, -1))
    out2 = _sc_fuse(packed, item2, brand2p, cat2p, pv, n,
                    brand_table.shape[0], cat_table.shape[0])
    return out2.reshape(n, D_ITEM)


# speedup vs baseline: 1.4309x; 1.4309x over previous
import functools

import jax
import jax.numpy as jnp
from jax import lax
from jax.experimental import pallas as pl
from jax.experimental.pallas import tpu as pltpu
from jax.experimental.pallas import tpu_sc as plsc

D_ITEM = 64
D_OTHER = 16
_CH = 256
_HC = 128


def _item_transform(item_table, wfi):
    V = item_table.shape[0]
    BR = 2048
    grid = (V + BR - 1) // BR

    def body(t_ref, w_ref, o_ref):
        o_ref[...] = _pack_cols(
            lax.dot_general(t_ref[...], w_ref[...], (((1,), (1,)), ((), ())),
                            preferred_element_type=jnp.float32))

    return pl.pallas_call(
        body,
        grid=(grid,),
        in_specs=[pl.BlockSpec((BR, D_ITEM), lambda i: (i, 0)),
                  pl.BlockSpec((D_ITEM, D_ITEM), lambda i: (0, 0))],
        out_specs=pl.BlockSpec((BR, 32), lambda i: (i, 0)),
        out_shape=jax.ShapeDtypeStruct((V, 32), jnp.int32),
    )(item_table, wfi)


def _pack_cols(m):
    lo = jnp.concatenate([m[:, 0:16], m[:, 32:48]], axis=1)
    hi = jnp.concatenate([m[:, 16:32], m[:, 48:64]], axis=1)
    lo16 = lax.bitcast_convert_type(lo.astype(jnp.bfloat16), jnp.uint16)
    hi16 = lax.bitcast_convert_type(hi.astype(jnp.bfloat16), jnp.uint16)
    word = lo16.astype(jnp.uint32) | (hi16.astype(jnp.uint32) << 16)
    return lax.bitcast_convert_type(word, jnp.int32)


def _small_transforms(brand_table, cat_table, wfb, wfc, wfp, Wp, bp2, bf2):
    def body(bt, ct, wb, wc, wpf, wpp, bpr, bfr, ob, oc, opv):
        b2 = lax.dot_general(bt[...], wb[...], (((1,), (1,)), ((), ())),
                             preferred_element_type=jnp.float32)
        ob[...] = _pack_cols(b2)
        c = lax.dot_general(bpr[...], wpf[...], (((1,), (1,)), ((), ())),
                            preferred_element_type=jnp.float32) + bfr[...]
        c2 = lax.dot_general(ct[...], wc[...], (((1,), (1,)), ((), ())),
                             preferred_element_type=jnp.float32) + c
        oc[...] = _pack_cols(c2)
        opv[...] = lax.dot_general(wpp[...], wpf[...], (((0,), (1,)), ((), ())),
                                   preferred_element_type=jnp.float32)

    nb = brand_table.shape[0]
    nc = cat_table.shape[0]
    return pl.pallas_call(
        body,
        out_shape=[jax.ShapeDtypeStruct((nb, 32), jnp.int32),
                   jax.ShapeDtypeStruct((nc, 32), jnp.int32),
                   jax.ShapeDtypeStruct((1, D_ITEM), jnp.float32)],
    )(brand_table, cat_table, wfb, wfc, wfp, Wp, bp2, bf2)


def _sc_fuse(packed, item2, brand2p, cat2p, pv, n, nb, nc):
    info = plsc.get_sparse_core_info()
    nw = info.num_cores * info.num_subcores
    assert n % (nw * _CH) == 0
    rows_w = n // nw
    nch = rows_w // _CH
    assert nch % 2 == 0
    npairs = nch // 2
    mesh = plsc.VectorSubcoreMesh(core_axis_name="c", subcore_axis_name="s")

    @functools.partial(
        pl.kernel, mesh=mesh,
        compiler_params=pltpu.CompilerParams(use_tc_tiling_on_sc=False,
                                             needs_layout_passes=False),
        out_type=jax.ShapeDtypeStruct((n, D_ITEM), jnp.float32),
        scratch_types=[
            pltpu.VMEM((8, _HC), jnp.int32),
            pltpu.VMEM((8, _HC), jnp.int32),
            pltpu.VMEM((_CH, 32), jnp.int32),
            pltpu.VMEM((_CH, 32), jnp.int32),
            pltpu.VMEM((_CH, D_ITEM), jnp.float32),
            pltpu.VMEM((_CH, D_ITEM), jnp.float32),
            pltpu.VMEM((nb, 32), jnp.int32),
            pltpu.VMEM((nc, 32), jnp.int32),
            pltpu.VMEM((D_ITEM,), jnp.float32),
            pltpu.SemaphoreType.DMA,
            pltpu.SemaphoreType.DMA,
            pltpu.SemaphoreType.DMA,
            pltpu.SemaphoreType.DMA,
            pltpu.SemaphoreType.DMA,
            pltpu.SemaphoreType.DMA,
        ],
    )
    def k(packed_hbm, it2, br2_hbm, ct2_hbm, pv_hbm, out_hbm,
          xb0, xb1, a0, a1, o0, o1, br_v, ct_v, pv_v,
          si0, si1, sg0, sg1, so0, so1):
        c16 = [jnp.full((16,), r, jnp.int32) for r in range(16)]
        col16 = [jnp.arange(16, dtype=jnp.int32) + 16 * w for w in range(2)]
        wid = lax.axis_index("s") * info.num_cores + lax.axis_index("c")
        cbase = wid * nch
        rbase = wid * rows_w
        pltpu.sync_copy(pv_hbm.at[0], pv_v)
        pltpu.sync_copy(br2_hbm, br_v)
        pltpu.sync_copy(ct2_hbm, ct_v)
        xb = (xb0, xb1)
        A = (a0, a1)
        O = (o0, o1)
        si = (si0, si1)
        sg = (sg0, sg1)
        so = (so0, so1)

        def fire_idx(g, s):
            pltpu.async_copy(packed_hbm.at[cbase + g], xb[s], si[s])

        def wait_idx(s):
            pltpu.make_async_copy(packed_hbm.at[0], xb[s], si[s]).wait()

        def fire_gathers(g, s):
            for h in range(2):
                dst = pl.ds(h * _HC, _HC)
                pltpu.async_copy(it2.at[xb[s].at[h]], A[s].at[dst], sg[s])

        def wait_gathers(s):
            for h in range(2):
                dst = pl.ds(h * _HC, _HC)
                pltpu.make_async_copy(it2.at[pl.ds(0, _HC)], A[s].at[dst],
                                      sg[s]).wait()

        def fire_store(g, s):
            pltpu.async_copy(O[s], out_hbm.at[pl.ds(rbase + g * _CH, _CH)],
                             so[s])

        def wait_store(s):
            pltpu.make_async_copy(O[s], out_hbm.at[pl.ds(0, _CH)],
                                  so[s]).wait()

        pvs_slices = [pl.ds(t * 16, 16) for t in range(4)]

        def gather_packed(tab, rowb):
            out = []
            for w in range(2):
                words = plsc.load_gather(tab, [rowb, col16[w]])
                s0, s1 = plsc.unpack(plsc.bitcast(words, jnp.bfloat16),
                                     format=plsc.PackFormat.INTERLEAVED)
                out.append(s0)
                out.append(s1)
            return out

        def alu(s):
            av, ov, xv = A[s], O[s], xb[s]
            pvs = [pv_v[sl] for sl in pvs_slices]

            def grp(j, carry2):
                jhi = j // 8
                jlo = pl.ds((j % 8) * 16, 16)
                bi_vec = xv[2 + jhi, jlo]
                ci_vec = xv[4 + jhi, jlo]
                pr_vec = plsc.bitcast(xv[6 + jhi, jlo], jnp.float32)
                for r in range(16):
                    nr = j * 16 + r
                    rb = bi_vec.at[c16[r]].get(mode="promise_in_bounds")
                    rc = ci_vec.at[c16[r]].get(mode="promise_in_bounds")
                    pb = pr_vec.at[c16[r]].get(mode="promise_in_bounds")
                    bs = gather_packed(br_v, rb)
                    cs = gather_packed(ct_v, rc)
                    ws = []
                    for w in range(2):
                        i0, i1 = plsc.unpack(
                            plsc.bitcast(av[nr, pl.ds(16 * w, 16)],
                                         jnp.bfloat16),
                            format=plsc.PackFormat.INTERLEAVED)
                        ws.append(i0)
                        ws.append(i1)
                    for t in range(4):
                        sl = pvs_slices[t]
                        ov[nr, sl] = (ws[t] + bs[t] + cs[t]
                                      + pb * pvs[t])
                return carry2

            lax.fori_loop(0, _CH // 16, grp, 0)

        fire_idx(0, 0)
        fire_idx(1, 1)
        wait_idx(0)
        fire_gathers(0, 0)

        def pair(p, carry):
            g = 2 * p
            wait_gathers(0)

            @pl.when(p > 0)
            def _():
                wait_store(1)

            wait_idx(1)
            fire_gathers(g + 1, 1)
            alu(0)
            fire_store(g, 0)

            @pl.when(p < npairs - 1)
            def _():
                fire_idx(g + 2, 0)

            wait_gathers(1)
            wait_store(0)

            @pl.when(p < npairs - 1)
            def _():
                wait_idx(0)
                fire_gathers(g + 2, 0)

            alu(1)
            fire_store(g + 1, 1)

            @pl.when(p < npairs - 1)
            def _():
                fire_idx(g + 3, 1)

            return carry

        lax.fori_loop(0, npairs, pair, 0)
        wait_store(1)

    return k(packed, item2, brand2p, cat2p, pv)


def kernel(x, item_table, brand_table, cat_table, Wp, bp, Wf, bf):
    n = x.shape[0]
    nchunks = n // _CH
    ii = x[:, 0].astype(jnp.int32).reshape(nchunks, 2, _HC)
    bi = x[:, 1].astype(jnp.int32).reshape(nchunks, 2, _HC)
    ci = x[:, 2].astype(jnp.int32).reshape(nchunks, 2, _HC)
    pb = lax.bitcast_convert_type(x[:, 3], jnp.int32).reshape(nchunks, 2, _HC)
    packed = jnp.concatenate([ii, bi, ci, pb], axis=1)
    wfi = Wf[:, :D_ITEM]
    wfb = Wf[:, D_ITEM:D_ITEM + D_OTHER]
    wfc = Wf[:, D_ITEM + D_OTHER:D_ITEM + 2 * D_OTHER]
    wfp = Wf[:, D_ITEM + 2 * D_OTHER:]
    item2 = _item_transform(item_table, wfi)
    brand2p, cat2p, pv = _small_transforms(
        brand_table, cat_table, wfb, wfc, wfp, Wp,
        bp.reshape(1, -1), bf.reshape(1, -1))
    return _sc_fuse(packed, item2, brand2p, cat2p, pv, n,
                    brand_table.shape[0], cat_table.shape[0])
